# Initial kernel scaffold; baseline (speedup 1.0000x reference)
import jax
import jax.numpy as jnp
from jax.experimental import pallas as pl


def kernel(x, emb_table, W, b):
    raise NotImplementedError("write your pallas kernel here")


# speedup vs baseline: 2.9212x; 2.9212x over previous
import functools

import jax
import jax.numpy as jnp
from jax import lax
from jax.experimental import pallas as pl
from jax.experimental.pallas import tpu as pltpu
from jax.experimental.pallas import tpu_sc as plsc

_V = 100000
_E = 64
_O = 128
_B = 16384
_F = 26
_BF = _B * _F

_ROW_BLK = 5000

_NC = 2
_NS = 16
_NW = _NC * _NS
_B_PER_W = _BF // _NW
_CHUNK = 128
_NCHUNK = _B_PER_W // _CHUNK
_NBUF = 4


def _proj_body(tab_ref, w_ref, b_ref, out_ref):
    acc = lax.dot_general(
        tab_ref[:], w_ref[:],
        dimension_numbers=(((1,), (1,)), ((), ())),
        preferred_element_type=jnp.float32,
    )
    out_ref[:] = jnp.maximum(acc + b_ref[:], 0.0)


def _project_table(emb_table, W, b2d):
    grid = (_V // _ROW_BLK,)
    return pl.pallas_call(
        _proj_body,
        grid=grid,
        in_specs=[
            pl.BlockSpec((_ROW_BLK, _E), lambda i: (i, 0)),
            pl.BlockSpec((_O, _E), lambda i: (0, 0)),
            pl.BlockSpec((1, _O), lambda i: (0, 0)),
        ],
        out_specs=pl.BlockSpec((_ROW_BLK, _O), lambda i: (i, 0)),
        out_shape=jax.ShapeDtypeStruct((_V, _O), jnp.float32),
    )(emb_table, W, b2d)


_sc_mesh = plsc.VectorSubcoreMesh(core_axis_name="c", subcore_axis_name="s")


@functools.partial(
    pl.kernel,
    mesh=_sc_mesh,
    out_type=jax.ShapeDtypeStruct((_BF, _O), jnp.float32),
    scratch_types=[
        pltpu.VMEM((_NCHUNK, _CHUNK), jnp.int32),
        pltpu.VMEM((_NBUF, _CHUNK, _O), jnp.float32),
        pltpu.SemaphoreType.DMA,
        pltpu.SemaphoreType.DMA,
    ],
)
def _sc_gather(q_hbm, idx_hbm, out_hbm, idx_v, rows_v, gsem, ssem):
    wid = lax.axis_index("s") * _NC + lax.axis_index("c")
    base = wid * _B_PER_W
    pltpu.sync_copy(idx_hbm.at[wid], idx_v)

    def outer(jo, carry):
        j0 = jo * _NBUF
        gathers = []
        for bi in range(_NBUF):
            gathers.append(
                pltpu.async_copy(q_hbm.at[idx_v.at[j0 + bi]], rows_v.at[bi], gsem)
            )
        scatters = []
        for bi in range(_NBUF):
            gathers[bi].wait()
            dst = out_hbm.at[pl.ds(base + (j0 + bi) * _CHUNK, _CHUNK)]
            scatters.append(pltpu.async_copy(rows_v.at[bi], dst, ssem))
        for s in scatters:
            s.wait()
        return carry

    lax.fori_loop(0, _NCHUNK // _NBUF, outer, 0)


def kernel(x, emb_table, W, b):
    q = _project_table(emb_table, W, b.reshape(1, _O))
    idx = x.astype(jnp.int32).reshape(_NW, _NCHUNK, _CHUNK)
    out = _sc_gather(q, idx)
    return out.reshape(_B, _F, _O)


# speedup vs baseline: 4.5543x; 1.5591x over previous
import functools

import jax
import jax.numpy as jnp
from jax import lax
from jax.experimental import pallas as pl
from jax.experimental.pallas import tpu as pltpu
from jax.experimental.pallas import tpu_sc as plsc

_V = 100000
_E = 64
_O = 128
_B = 16384
_F = 26
_BF = _B * _F

_ROW_BLK = 5000

_NC = 2
_NS = 16
_NW = _NC * _NS
_B_PER_W = _B // _NW
_SLABS = 4
_CHUNK = _SLABS * _F
_NCHUNK = _B_PER_W // _SLABS
_NBUF = 4


def _proj_body(tab_ref, w_ref, b_ref, out_ref):
    acc = lax.dot_general(
        tab_ref[:], w_ref[:],
        dimension_numbers=(((1,), (1,)), ((), ())),
        preferred_element_type=jnp.float32,
    )
    out_ref[:] = jnp.maximum(acc + b_ref[:], 0.0)


def _project_table(emb_table, W, b2d):
    grid = (_V // _ROW_BLK,)
    return pl.pallas_call(
        _proj_body,
        grid=grid,
        in_specs=[
            pl.BlockSpec((_ROW_BLK, _E), lambda i: (i, 0)),
            pl.BlockSpec((_O, _E), lambda i: (0, 0)),
            pl.BlockSpec((1, _O), lambda i: (0, 0)),
        ],
        out_specs=pl.BlockSpec((_ROW_BLK, _O), lambda i: (i, 0)),
        out_shape=jax.ShapeDtypeStruct((_V, _O), jnp.float32),
    )(emb_table, W, b2d)


_sc_mesh = plsc.VectorSubcoreMesh(core_axis_name="c", subcore_axis_name="s")


@functools.partial(
    pl.kernel,
    mesh=_sc_mesh,
    out_type=jax.ShapeDtypeStruct((_B, _F, _O), jnp.float32),
    scratch_types=[
        pltpu.VMEM((_NCHUNK, _CHUNK), jnp.int32),
        pltpu.VMEM((_NBUF, _CHUNK, _O), jnp.float32),
        pltpu.SemaphoreType.DMA,
        pltpu.SemaphoreType.DMA,
    ],
)
def _sc_gather(q_hbm, idx_hbm, out_hbm, idx_v, rows_v, gsem, ssem):
    wid = lax.axis_index("s") * _NC + lax.axis_index("c")
    base = wid * _B_PER_W
    pltpu.sync_copy(idx_hbm.at[wid], idx_v)

    def outer(jo, carry):
        j0 = jo * _NBUF
        gathers = []
        for bi in range(_NBUF):
            gathers.append(
                pltpu.async_copy(q_hbm.at[idx_v.at[j0 + bi]], rows_v.at[bi], gsem)
            )
        scatters = []
        for bi in range(_NBUF):
            gathers[bi].wait()
            b0 = base + (j0 + bi) * _SLABS
            for s in range(_SLABS):
                src = rows_v.at[bi, pl.ds(s * _F, _F)]
                scatters.append(pltpu.async_copy(src, out_hbm.at[b0 + s], ssem))
        for s in scatters:
            s.wait()
        return carry

    lax.fori_loop(0, _NCHUNK // _NBUF, outer, 0)


def kernel(x, emb_table, W, b):
    q = _project_table(emb_table, W, b.reshape(1, _O))
    idx = x.astype(jnp.int32).reshape(_NW, _NCHUNK, _CHUNK)
    return _sc_gather(q, idx)
